# baseline (device time: 67717 ns/iter reference)
import jax
import jax.numpy as jnp
from jax import lax
from jax.experimental import pallas as pl
from jax.experimental.pallas import tpu as pltpu

N_X = 2


def kernel(O, Wo):
    B, S, H, D = O.shape
    K = H * D
    N = Wo.shape[1]
    S_half = S // N_X

    O3 = O.reshape(B, S, K)

    def body(o_ref, w_ref, out_ref, send_buf, recv_buf, send_sem, recv_sem):
        my_x = lax.axis_index("x")
        my_y = lax.axis_index("y")
        peer = (1 - my_x, my_y)

        barrier_sem = pltpu.get_barrier_semaphore()
        pl.semaphore_signal(
            barrier_sem, inc=1,
            device_id=peer, device_id_type=pl.DeviceIdType.MESH,
        )
        pl.semaphore_wait(barrier_sem, 1)

        w_bf = w_ref[...].astype(jnp.bfloat16)

        peer_start = (1 - my_x) * S_half
        for b in range(B):
            o_bf = o_ref[b, pl.ds(peer_start, S_half), :].astype(jnp.bfloat16)
            send_buf[b, :, :] = jnp.dot(
                o_bf, w_bf, preferred_element_type=jnp.float32
            ).astype(jnp.bfloat16)

        rdma = pltpu.make_async_remote_copy(
            src_ref=send_buf,
            dst_ref=recv_buf,
            send_sem=send_sem,
            recv_sem=recv_sem,
            device_id=peer,
            device_id_type=pl.DeviceIdType.MESH,
        )
        rdma.start()

        my_start = my_x * S_half
        for b in range(B):
            o_bf = o_ref[b, pl.ds(my_start, S_half), :].astype(jnp.bfloat16)
            out_ref[b, :, :] = jnp.dot(
                o_bf, w_bf, preferred_element_type=jnp.float32
            )

        rdma.wait()

        for b in range(B):
            out_ref[b, :, :] += recv_buf[b, :, :].astype(jnp.float32)

    return pl.pallas_call(
        body,
        out_shape=jax.ShapeDtypeStruct((B, S_half, N), jnp.float32),
        in_specs=[
            pl.BlockSpec(memory_space=pltpu.VMEM),
            pl.BlockSpec(memory_space=pltpu.VMEM),
        ],
        out_specs=pl.BlockSpec(memory_space=pltpu.VMEM),
        scratch_shapes=[
            pltpu.VMEM((B, S_half, N), jnp.bfloat16),
            pltpu.VMEM((B, S_half, N), jnp.bfloat16),
            pltpu.SemaphoreType.DMA,
            pltpu.SemaphoreType.DMA,
        ],
        compiler_params=pltpu.CompilerParams(collective_id=0),
    )(O3, Wo)


# device time: 49204 ns/iter; 1.3762x vs baseline; 1.3762x over previous
import jax
import jax.numpy as jnp
from jax import lax
from jax.experimental import pallas as pl
from jax.experimental.pallas import tpu as pltpu

N_X = 2
CHUNKS_PER_B = 2


def kernel(O, Wo):
    B, S, H, D = O.shape
    K = H * D
    N = Wo.shape[1]
    S_half = S // N_X
    R = S_half // CHUNKS_PER_B
    NCHUNK = B * CHUNKS_PER_B

    O3 = O.reshape(B, S, K)

    def body(o_ref, w_ref, out_ref, send_buf, recv_buf, send_sems, recv_sems):
        my_x = lax.axis_index("x")
        my_y = lax.axis_index("y")
        peer = (1 - my_x, my_y)

        barrier_sem = pltpu.get_barrier_semaphore()
        pl.semaphore_signal(
            barrier_sem, inc=1,
            device_id=peer, device_id_type=pl.DeviceIdType.MESH,
        )
        pl.semaphore_wait(barrier_sem, 1)

        w_bf = w_ref[...].astype(jnp.bfloat16)
        peer_start = (1 - my_x) * S_half
        my_start = my_x * S_half

        rdmas = []
        for c in range(NCHUNK):
            b, row = divmod(c, CHUNKS_PER_B)
            o_bf = o_ref[b, pl.ds(peer_start + row * R, R), :].astype(
                jnp.bfloat16
            )
            send_buf[c, :, :] = jnp.dot(
                o_bf, w_bf, preferred_element_type=jnp.float32
            ).astype(jnp.bfloat16)
            rdma = pltpu.make_async_remote_copy(
                src_ref=send_buf.at[c],
                dst_ref=recv_buf.at[c],
                send_sem=send_sems.at[c],
                recv_sem=recv_sems.at[c],
                device_id=peer,
                device_id_type=pl.DeviceIdType.MESH,
            )
            rdma.start()
            rdmas.append(rdma)

        for b in range(B):
            o_bf = o_ref[b, pl.ds(my_start, S_half), :].astype(jnp.bfloat16)
            out_ref[b, :, :] = jnp.dot(
                o_bf, w_bf, preferred_element_type=jnp.float32
            )

        for c in range(NCHUNK):
            b, row = divmod(c, CHUNKS_PER_B)
            rdmas[c].wait_recv()
            out_ref[b, row * R:(row + 1) * R, :] += recv_buf[c].astype(
                jnp.float32
            )

        for c in range(NCHUNK):
            rdmas[c].wait_send()

    return pl.pallas_call(
        body,
        out_shape=jax.ShapeDtypeStruct((B, S_half, N), jnp.float32),
        in_specs=[
            pl.BlockSpec(memory_space=pltpu.VMEM),
            pl.BlockSpec(memory_space=pltpu.VMEM),
        ],
        out_specs=pl.BlockSpec(memory_space=pltpu.VMEM),
        scratch_shapes=[
            pltpu.VMEM((NCHUNK, R, N), jnp.bfloat16),
            pltpu.VMEM((NCHUNK, R, N), jnp.bfloat16),
            pltpu.SemaphoreType.DMA((NCHUNK,)),
            pltpu.SemaphoreType.DMA((NCHUNK,)),
        ],
        compiler_params=pltpu.CompilerParams(collective_id=0),
    )(O3, Wo)
